# baseline (speedup 1.0000x reference)
import functools

import jax
import jax.numpy as jnp
from jax import lax
from jax.experimental import pallas as pl
from jax.experimental.pallas import tpu as pltpu
from jax.experimental.pallas import tpu_sc as plsc

B = 16384
F = 26
L = 20
D = 32
NW = 32
ROWS_PER_W = B // NW
BAGS_PER_W = ROWS_PER_W * F
IDX_PER_W = BAGS_PER_W * L
CHUNK_BAGS = 32
CHUNK_IDX = CHUNK_BAGS * L
N_STREAMS = CHUNK_IDX // 128
N_CHUNKS = IDX_PER_W // CHUNK_IDX
IDX_BATCH = 16


def _sc_pool(x_resh, table):
    mesh = plsc.VectorSubcoreMesh(core_axis_name="c", subcore_axis_name="s")

    @functools.partial(
        pl.kernel,
        mesh=mesh,
        out_type=jax.ShapeDtypeStruct((B * F, D), jnp.bfloat16),
        scratch_types=[
            pltpu.VMEM((IDX_BATCH, CHUNK_IDX), jnp.int32),
            pltpu.VMEM((CHUNK_IDX, D), jnp.bfloat16),
            pltpu.VMEM((CHUNK_IDX, D), jnp.bfloat16),
            pltpu.VMEM((CHUNK_BAGS, D), jnp.bfloat16),
            pltpu.VMEM((CHUNK_BAGS, D), jnp.bfloat16),
            pltpu.SemaphoreType.DMA,
            pltpu.SemaphoreType.DMA,
            pltpu.SemaphoreType.DMA,
            pltpu.SemaphoreType.DMA,
        ],
        compiler_params=pltpu.CompilerParams(use_tc_tiling_on_sc=False,
                                             needs_layout_passes=False),
    )
    def body(x_hbm, table_hbm, out_hbm, idx_v, rows0, rows1, outv0, outv1,
             gsem0, gsem1, osem0, osem1):
        wid = lax.axis_index("s") * 2 + lax.axis_index("c")
        rows = (rows0, rows1)
        outv = (outv0, outv1)
        gsem = (gsem0, gsem1)
        osem = (osem0, osem1)

        def fire(c, buf):
            c16 = c % IDX_BATCH
            pltpu.async_copy(
                table_hbm.at[idx_v.at[c16]],
                rows[buf],
                gsem[buf],
            )

        def drain_gather(buf):
            pltpu.make_async_copy(
                table_hbm.at[idx_v.at[0]],
                rows[buf],
                gsem[buf],
            ).wait()

        def stage_idx(c):
            pltpu.sync_copy(x_hbm.at[wid, c // IDX_BATCH], idx_v)

        stage_idx(0)
        fire(0, 0)

        def pair_body(g, carry):
            for b in range(2):
                c = 2 * g + b
                nb = 1 - b

                drain_gather(b)

                @pl.when(jnp.logical_and(c % IDX_BATCH == IDX_BATCH - 1,
                                         c + 1 < N_CHUNKS))
                def _():
                    stage_idx(c + 1)

                @pl.when(c + 1 < N_CHUNKS)
                def _():
                    fire(c + 1, nb)

                @pl.when(c >= 2)
                def _():
                    pltpu.make_async_copy(
                        outv[b],
                        out_hbm.at[pl.ds(0, CHUNK_BAGS)],
                        osem[b],
                    ).wait()

                def bag_body(bb, carry2):
                    base = bb * L
                    us, vs = [], []
                    for k in range(L):
                        u, v = plsc.unpack(
                            rows[b][base + k, pl.ds(0, 2 * 16)],
                            format=plsc.PackFormat.INTERLEAVED,
                            preferred_element_type=jnp.float32,
                        )
                        us.append(u)
                        vs.append(v)
                    for t in (us, vs):
                        while len(t) > 1:
                            t[:] = [t[i] + t[i + 1]
                                    for i in range(0, len(t) - 1, 2)] \
                                + ([t[-1]] if len(t) % 2 else [])
                    outv[b][bb, pl.ds(0, 2 * 16)] = plsc.pack(
                        us[0], vs[0],
                        format=plsc.PackFormat.INTERLEAVED,
                    )
                    return carry2

                lax.fori_loop(0, CHUNK_BAGS, bag_body, 0)

                pltpu.async_copy(
                    outv[b],
                    out_hbm.at[pl.ds(wid * BAGS_PER_W + c * CHUNK_BAGS,
                                     CHUNK_BAGS)],
                    osem[b],
                )
            return carry

        lax.fori_loop(0, N_CHUNKS // 2, pair_body, 0)

        for b in range(2):
            pltpu.make_async_copy(
                outv[b], out_hbm.at[pl.ds(0, CHUNK_BAGS)], osem[b]
            ).wait()

    return body(x_resh, table)


def _tc_normalize(flat):
    BR = 1024

    def body(x_ref, o_ref):
        x = x_ref[...].astype(jnp.float32)
        s = jnp.sum(x * x, axis=1, keepdims=True)
        o_ref[...] = x * lax.rsqrt(jnp.maximum(s, 1e-24))

    return pl.pallas_call(
        body,
        out_shape=jax.ShapeDtypeStruct((B, F * D), jnp.float32),
        grid=(B // BR,),
        in_specs=[pl.BlockSpec((BR, F * D), lambda i: (i, 0))],
        out_specs=pl.BlockSpec((BR, F * D), lambda i: (i, 0)),
    )(flat)


def kernel(x, table):
    x_resh = x.reshape(NW, N_CHUNKS // IDX_BATCH, IDX_BATCH, CHUNK_IDX)
    pooled = _sc_pool(x_resh, table.astype(jnp.bfloat16))
    flat = pooled.reshape(B, F * D)
    return _tc_normalize(flat)


# speedup vs baseline: 1.2272x; 1.2272x over previous
import functools

import jax
import jax.numpy as jnp
from jax import lax
from jax.experimental import pallas as pl
from jax.experimental.pallas import tpu as pltpu
from jax.experimental.pallas import tpu_sc as plsc

B = 16384
F = 26
L = 20
D = 32
NW = 32
ROWS_PER_W = B // NW
BAGS_PER_W = ROWS_PER_W * F
IDX_PER_W = BAGS_PER_W * L
CHUNK_BAGS = 64
CHUNK_IDX = CHUNK_BAGS * L
N_STREAMS = CHUNK_IDX // 128
N_CHUNKS = IDX_PER_W // CHUNK_IDX
IDX_BATCH = 8


def _sc_pool(x_resh, table):
    mesh = plsc.VectorSubcoreMesh(core_axis_name="c", subcore_axis_name="s")

    @functools.partial(
        pl.kernel,
        mesh=mesh,
        out_type=jax.ShapeDtypeStruct((B * F, D), jnp.float32),
        scratch_types=[
            pltpu.VMEM((IDX_BATCH, CHUNK_IDX), jnp.int32),
            pltpu.VMEM((CHUNK_IDX, D), jnp.float32),
            pltpu.VMEM((CHUNK_IDX, D), jnp.float32),
            pltpu.VMEM((CHUNK_BAGS, D), jnp.float32),
            pltpu.VMEM((CHUNK_BAGS, D), jnp.float32),
            pltpu.SemaphoreType.DMA,
            pltpu.SemaphoreType.DMA,
            pltpu.SemaphoreType.DMA,
            pltpu.SemaphoreType.DMA,
        ],
        compiler_params=pltpu.CompilerParams(use_tc_tiling_on_sc=False),
    )
    def body(x_hbm, table_hbm, out_hbm, idx_v, rows0, rows1, outv0, outv1,
             gsem0, gsem1, osem0, osem1):
        wid = lax.axis_index("s") * 2 + lax.axis_index("c")
        rows = (rows0, rows1)
        outv = (outv0, outv1)
        gsem = (gsem0, gsem1)
        osem = (osem0, osem1)

        def fire(c, buf):
            c16 = c % IDX_BATCH
            pltpu.async_copy(
                table_hbm.at[idx_v.at[c16]],
                rows[buf],
                gsem[buf],
            )

        def drain_gather(buf):
            pltpu.make_async_copy(
                table_hbm.at[idx_v.at[0]],
                rows[buf],
                gsem[buf],
            ).wait()

        def stage_idx(c):
            pltpu.sync_copy(x_hbm.at[wid, c // IDX_BATCH], idx_v)

        stage_idx(0)
        fire(0, 0)

        def pair_body(g, carry):
            for b in range(2):
                c = 2 * g + b
                nb = 1 - b

                drain_gather(b)

                @pl.when(jnp.logical_and(c % IDX_BATCH == IDX_BATCH - 1,
                                         c + 1 < N_CHUNKS))
                def _():
                    stage_idx(c + 1)

                @pl.when(c + 1 < N_CHUNKS)
                def _():
                    fire(c + 1, nb)

                @pl.when(c >= 2)
                def _():
                    pltpu.make_async_copy(
                        outv[b],
                        out_hbm.at[pl.ds(0, CHUNK_BAGS)],
                        osem[b],
                    ).wait()

                def bag_body(bb, carry2):
                    base = bb * L
                    for h in range(2):
                        t = []
                        for k in range(L):
                            r = base + k
                            t.append(rows[b][r, pl.ds(16 * h, 16)])
                        while len(t) > 1:
                            t = [t[i] + t[i + 1] for i in range(0, len(t) - 1, 2)] \
                                + ([t[-1]] if len(t) % 2 else [])
                        outv[b][bb, pl.ds(16 * h, 16)] = t[0]
                    return carry2

                @plsc.parallel_loop(0, CHUNK_BAGS, 1, unroll=2)
                def _(bb):
                    bag_body(bb, 0)

                pltpu.async_copy(
                    outv[b],
                    out_hbm.at[pl.ds(wid * BAGS_PER_W + c * CHUNK_BAGS,
                                     CHUNK_BAGS)],
                    osem[b],
                )
            return carry

        lax.fori_loop(0, N_CHUNKS // 2, pair_body, 0)

        for b in range(2):
            pltpu.make_async_copy(
                outv[b], out_hbm.at[pl.ds(0, CHUNK_BAGS)], osem[b]
            ).wait()

    return body(x_resh, table)


def _tc_normalize(flat):
    BR = 1024

    def body(x_ref, o_ref):
        x = x_ref[...]
        s = jnp.sum(x * x, axis=1, keepdims=True)
        o_ref[...] = x * lax.rsqrt(jnp.maximum(s, 1e-24))

    return pl.pallas_call(
        body,
        out_shape=jax.ShapeDtypeStruct((B, F * D), jnp.float32),
        grid=(B // BR,),
        in_specs=[pl.BlockSpec((BR, F * D), lambda i: (i, 0))],
        out_specs=pl.BlockSpec((BR, F * D), lambda i: (i, 0)),
    )(flat)


def kernel(x, table):
    x_resh = x.reshape(NW, N_CHUNKS // IDX_BATCH, IDX_BATCH, CHUNK_IDX)
    pooled = _sc_pool(x_resh, table)
    flat = pooled.reshape(B, F * D)
    return _tc_normalize(flat)
